# Initial kernel scaffold; baseline (speedup 1.0000x reference)
import jax
import jax.numpy as jnp
from jax.experimental import pallas as pl


def kernel(feature, W, b, gamma, beta):
    raise NotImplementedError("write your pallas kernel here")


# speedup vs baseline: 3.0825x; 3.0825x over previous
import functools

import jax
import jax.numpy as jnp
from jax.experimental import pallas as pl
from jax.experimental.pallas import tpu as pltpu

TOPK = 300
H = 512
W_ = 512
C = 96
NTAP = 9
HW = H * W_
BS = 16384


def _contract_body(w_ref, f_ref, g_ref):
    fb = f_ref[...].astype(jnp.bfloat16)
    g = jax.lax.dot_general(
        w_ref[...], fb, (((1,), (0,)), ((), ())),
        preferred_element_type=jnp.float32,
    )
    g_ref[...] = g[:NTAP, :]


def _stencil_body(g_ref, s_ref, p_ref):
    zrow = jnp.zeros((1, W_), jnp.float32)
    zcol = jnp.zeros((H, 1), jnp.float32)

    def shift(src, dy, dx):
        if dy == 0:
            src = jnp.concatenate([zrow, src[: H - 1, :]], axis=0)
        elif dy == 2:
            src = jnp.concatenate([src[1:, :], zrow], axis=0)
        if dx == 0:
            src = jnp.concatenate([zcol, src[:, : W_ - 1]], axis=1)
        elif dx == 2:
            src = jnp.concatenate([src[:, 1:], zcol], axis=1)
        return src

    acc = None
    for t in range(NTAP):
        dy, dx = t // 3, t % 3
        s = shift(g_ref[t], dy, dx)
        acc = s if acc is None else acc + s
    bv = s_ref[0]
    gv = s_ref[1]
    betav = s_ref[2]
    sq = s_ref[3]
    y = gv * (acc + bv) / sq + betav
    p_ref[...] = jax.nn.sigmoid(y)


def _conv_sigmoid(feature, W, b, gamma, beta):
    N = feature.shape[0]
    wb = W.astype(jnp.bfloat16)[0]
    wmat = wb.reshape(C, NTAP).T
    wmat = jnp.concatenate(
        [wmat, jnp.zeros((16 - NTAP, C), jnp.bfloat16)], axis=0
    )
    f_flat = feature.reshape(N, C, HW)

    g = pl.pallas_call(
        _contract_body,
        grid=(N, HW // BS),
        in_specs=[
            pl.BlockSpec((16, C), lambda n, j: (0, 0)),
            pl.BlockSpec((None, C, BS), lambda n, j: (n, 0, j)),
        ],
        out_specs=pl.BlockSpec((None, NTAP, BS), lambda n, j: (n, 0, j)),
        out_shape=jax.ShapeDtypeStruct((N, NTAP, HW), jnp.float32),
    )(wmat, f_flat)

    scal = jnp.concatenate(
        [
            b.astype(jnp.float32),
            gamma.astype(jnp.float32),
            beta.astype(jnp.float32),
            jnp.sqrt(jnp.float32(1.0 + 1e-5))[None],
        ]
    )

    g4 = g.reshape(N, NTAP, H, W_)
    p = pl.pallas_call(
        _stencil_body,
        grid=(N,),
        in_specs=[
            pl.BlockSpec((None, NTAP, H, W_), lambda n: (n, 0, 0, 0)),
            pl.BlockSpec(memory_space=pltpu.SMEM),
        ],
        out_specs=pl.BlockSpec((None, None, H, W_), lambda n: (n, 0, 0, 0)),
        out_shape=jax.ShapeDtypeStruct((N, 1, H, W_), jnp.float32),
    )(g4, scal)
    return p


def kernel(feature, W, b, gamma, beta):
    p = _conv_sigmoid(feature, W, b, gamma, beta)
    psq = p[:, 0, :, :]
    N = psq.shape[0]
    flat = psq.reshape(N, -1)
    values, indices = jax.lax.top_k(flat, TOPK)
    xy = jnp.stack(
        (
            (indices // W_).astype(jnp.float32) / float(H),
            (indices % W_).astype(jnp.float32) / float(W_),
        ),
        axis=-1,
    )
    return (p, values, xy, indices)


# speedup vs baseline: 5.2239x; 1.6947x over previous
import functools

import jax
import jax.numpy as jnp
from jax import lax
from jax.experimental import pallas as pl
from jax.experimental.pallas import tpu as pltpu
from jax.experimental.pallas import tpu_sc as plsc

TOPK = 300
H = 512
W_ = 512
C = 96
NTAP = 9
HW = H * W_
BS = 16384


def _contract_body(w_ref, f_ref, g_ref):
    fb = f_ref[...].astype(jnp.bfloat16)
    g = jax.lax.dot_general(
        w_ref[...], fb, (((1,), (0,)), ((), ())),
        preferred_element_type=jnp.float32,
    )
    g_ref[...] = g[:NTAP, :]


def _stencil_body(g_ref, s_ref, p_ref):
    zrow = jnp.zeros((1, W_), jnp.float32)
    zcol = jnp.zeros((H, 1), jnp.float32)

    def shift(src, dy, dx):
        if dy == 0:
            src = jnp.concatenate([zrow, src[: H - 1, :]], axis=0)
        elif dy == 2:
            src = jnp.concatenate([src[1:, :], zrow], axis=0)
        if dx == 0:
            src = jnp.concatenate([zcol, src[:, : W_ - 1]], axis=1)
        elif dx == 2:
            src = jnp.concatenate([src[:, 1:], zcol], axis=1)
        return src

    acc = None
    for t in range(NTAP):
        dy, dx = t // 3, t % 3
        s = shift(g_ref[t], dy, dx)
        acc = s if acc is None else acc + s
    bv = s_ref[0]
    gv = s_ref[1]
    betav = s_ref[2]
    sq = s_ref[3]
    y = gv * (acc + bv) / sq + betav
    p_ref[...] = jax.nn.sigmoid(y)


def _conv_sigmoid(feature, W, b, gamma, beta):
    N = feature.shape[0]
    wb = W.astype(jnp.bfloat16)[0]
    wmat = wb.reshape(C, NTAP).T
    wmat = jnp.concatenate(
        [wmat, jnp.zeros((16 - NTAP, C), jnp.bfloat16)], axis=0
    )
    f_flat = feature.reshape(N, C, HW)

    g = pl.pallas_call(
        _contract_body,
        grid=(N, HW // BS),
        in_specs=[
            pl.BlockSpec((16, C), lambda n, j: (0, 0)),
            pl.BlockSpec((None, C, BS), lambda n, j: (n, 0, j)),
        ],
        out_specs=pl.BlockSpec((None, NTAP, BS), lambda n, j: (n, 0, j)),
        out_shape=jax.ShapeDtypeStruct((N, NTAP, HW), jnp.float32),
    )(wmat, f_flat)

    scal = jnp.concatenate(
        [
            b.astype(jnp.float32),
            gamma.astype(jnp.float32),
            beta.astype(jnp.float32),
            jnp.sqrt(jnp.float32(1.0 + 1e-5))[None],
        ]
    )

    g4 = g.reshape(N, NTAP, H, W_)
    p = pl.pallas_call(
        _stencil_body,
        grid=(N,),
        in_specs=[
            pl.BlockSpec((None, NTAP, H, W_), lambda n: (n, 0, 0, 0)),
            pl.BlockSpec(memory_space=pltpu.SMEM),
        ],
        out_specs=pl.BlockSpec((None, None, H, W_), lambda n: (n, 0, 0, 0)),
        out_shape=jax.ShapeDtypeStruct((N, 1, H, W_), jnp.float32),
    )(g4, scal)
    return p


L = 16
NSUB = 16
CHUNK = HW // NSUB
NVEC = CHUNK // L
NBIN = 4096
NGRP = NBIN // L
SLABW = NBIN // NSUB
CAP_W = 512
CAP_G = 1024
OUTP = 320


def _sc_topk_body(p_hbm, oval_hbm, oidx_hbm,
                  chunk, hist, slab, comb_l, ck, ci, cnt_stage, cnt_l,
                  ak, ai, lv, li, zi, seqi,
                  sh16, comb0, comb1, comb2, shcnt, shsl_k, shsl_i,
                  shck, shci, shv, shi):
    n = lax.axis_index("c")
    sid = lax.axis_index("s")
    base = pl.multiple_of(n * HW + sid * CHUNK, CHUNK)
    iota = lax.iota(jnp.int32, L)
    zero_iv = jnp.zeros((L,), jnp.int32)
    one_iv = jnp.ones((L,), jnp.int32)

    def zero_ref(ref, nvec, val):
        def f(i, _):
            ref[pl.ds(i * L, L)] = val
            return 0
        lax.fori_loop(0, nvec, f, 0)

    zero_ref(ck, CAP_W // L, zero_iv)
    zero_ref(ci, CAP_W // L, zero_iv)
    zero_ref(ak, CAP_G // L, zero_iv)
    zero_ref(ai, CAP_G // L, zero_iv)
    zero_ref(lv, OUTP // L, zero_iv)
    zero_ref(li, OUTP // L, zero_iv)
    zero_ref(zi, OUTP // L, zero_iv)

    def fill_seq(i, _):
        seqi[pl.ds(i * L, L)] = iota + i * L
        return 0
    lax.fori_loop(0, OUTP // L, fill_seq, 0)

    pltpu.sync_copy(p_hbm.at[pl.ds(base, CHUNK)], chunk)

    def hist_pass(comb_sh, bin_fn, k_need):
        zero_ref(hist, NGRP, zero_iv)

        def acc(i, _):
            key = chunk[pl.ds(i * L, L)]
            bins, msk = bin_fn(key)
            plsc.addupdate_scatter(hist, [bins], one_iv, mask=msk)
            return 0
        lax.fori_loop(0, NVEC, acc, 0)

        pltpu.sync_copy(hist, sh16.at[sid])
        plsc.subcore_barrier()
        pltpu.sync_copy(sh16.at[:, pl.ds(sid * SLABW, SLABW)], slab)

        def mrg(j, _):
            a = jnp.zeros((L,), jnp.int32)
            for r in range(NSUB):
                a = a + slab[r, pl.ds(j * L, L)]
            comb_l[pl.ds(j * L, L)] = a
            return 0
        lax.fori_loop(0, SLABW // L, mrg, 0)
        pltpu.sync_copy(comb_l.at[pl.ds(0, SLABW)],
                        comb_sh.at[pl.ds(sid * SLABW, SLABW)])
        plsc.subcore_barrier()
        pltpu.sync_copy(comb_sh, comb_l)

        def scan(t, carry):
            cum, gsel, kloc = carry
            g = NGRP - 1 - t
            s = jnp.sum(comb_l[pl.ds(g * L, L)])
            newcum = cum + s
            hit = jnp.logical_and(cum < k_need, newcum >= k_need)
            gsel = jnp.where(hit, g, gsel)
            kloc = jnp.where(hit, k_need - cum, kloc)
            return (newcum, gsel, kloc)
        _, gsel, kloc = lax.fori_loop(0, NGRP, scan, (0, 0, 1))

        hv = comb_l[pl.ds(pl.multiple_of(gsel * L, L), L)]
        suf = lax.rev(plsc.cumsum(lax.rev(hv, (0,))), (0,))
        ge = suf >= kloc
        boff_v = plsc.all_reduce_population_count(ge) - 1
        above = jnp.sum(jnp.where(iota > boff_v, hv, 0))
        k_next = kloc - above
        b_v = jnp.full((L,), gsel * L, jnp.int32) + boff_v
        return b_v, k_next

    b0_v, k1 = hist_pass(
        comb0,
        lambda key: (lax.shift_right_logical(key, 19), None),
        300)
    b1_v, k2 = hist_pass(
        comb1,
        lambda key: (
            jnp.bitwise_and(lax.shift_right_logical(key, 7), 0xFFF),
            lax.shift_right_logical(key, 19) == b0_v),
        k1)
    pref2_v = jnp.bitwise_or(lax.shift_left(b0_v, 12), b1_v)
    b2_v, _ = hist_pass(
        comb2,
        lambda key: (
            jnp.bitwise_and(key, 0x7F),
            lax.shift_right_logical(key, 7) == pref2_v),
        k2)
    t_v = jnp.bitwise_or(
        jnp.bitwise_or(lax.shift_left(b0_v, 19), lax.shift_left(b1_v, 7)),
        b2_v)

    def extract(i, off):
        key = chunk[pl.ds(i * L, L)]
        m = jnp.logical_and(key >= t_v, off < CAP_W - L)
        pc = plsc.all_reduce_population_count(m)[0]
        pos = off + plsc.cumsum(jnp.where(m, 1, 0)) - 1
        plsc.store_scatter(ck, [pos], key, mask=m)
        gidx = sid * CHUNK + i * L + iota
        plsc.store_scatter(ci, [pos], gidx, mask=m)
        return off + pc
    off = lax.fori_loop(0, NVEC, extract, 0)
    cw16 = jnp.bitwise_and(off + (L - 1), ~(L - 1))

    cnt_stage[pl.ds(0, L)] = jnp.where(iota == 0, cw16, 0)
    pltpu.sync_copy(cnt_stage, shcnt.at[sid, pl.ds(0, L)])

    def slabwr(t, _):
        pltpu.sync_copy(ck.at[pl.ds(t * L, L)],
                        shsl_k.at[sid, pl.ds(t * L, L)])
        pltpu.sync_copy(ci.at[pl.ds(t * L, L)],
                        shsl_i.at[sid, pl.ds(t * L, L)])
        return 0
    lax.fori_loop(0, cw16 // L, slabwr, 0)

    @pl.when(sid == 0)
    def _():
        pltpu.sync_copy(zi, shv)
        pltpu.sync_copy(zi, shi)
    plsc.subcore_barrier()
    pltpu.sync_copy(shcnt, cnt_l)

    def tots(w, tot):
        rv = cnt_l[w, pl.ds(0, L)]
        return tot + rv[0]
    c_tot = jnp.minimum(lax.fori_loop(0, NSUB, tots, 0), CAP_G)

    @pl.when(sid == 0)
    def _():
        def outer(w, dst):
            rv = cnt_l[w, pl.ds(0, L)]
            cwk = rv[0]

            def inner(t, dst2):
                @pl.when(dst2 < CAP_G)
                def _():
                    d = pl.multiple_of(dst2, L)
                    pltpu.sync_copy(shsl_k.at[w, pl.ds(t * L, L)],
                                    ak.at[pl.ds(d, L)])
                    pltpu.sync_copy(shsl_i.at[w, pl.ds(t * L, L)],
                                    ai.at[pl.ds(d, L)])
                return dst2 + L
            return lax.fori_loop(0, cwk // L, inner, dst)
        lax.fori_loop(0, NSUB, outer, 0)
        pltpu.sync_copy(ak, shck)
        pltpu.sync_copy(ai, shci)
    plsc.subcore_barrier()

    pltpu.sync_copy(shck, ak)
    pltpu.sync_copy(shci, ai)
    c_tot_v = jnp.full((L,), c_tot, jnp.int32)
    for v in range(CAP_G // (NSUB * L)):
        slotbase = pl.multiple_of(sid * L + v * (NSUB * L), L)
        slots = jnp.full((L,), slotbase, jnp.int32) + iota
        valid = slots < c_tot_v
        km = jnp.where(valid, ak[pl.ds(slotbase, L)], 0)
        im = jnp.where(valid, ai[pl.ds(slotbase, L)], 0)

        def rankjv(jv, r):
            kvec = ak[pl.ds(jv * L, L)]
            ivec = ai[pl.ds(jv * L, L)]
            for lane in range(L):
                kjv = jnp.full((L,), kvec[lane], jnp.int32)
                ijv = jnp.full((L,), ivec[lane], jnp.int32)
                gt = kjv > km
                eq = jnp.logical_and(kjv == km, ijv < im)
                r = r + jnp.where(jnp.logical_or(gt, eq), 1, 0)
            return r
        r = lax.fori_loop(0, c_tot // L, rankjv, zero_iv)
        sel = jnp.logical_and(r < TOPK, valid)
        plsc.store_scatter(lv, [r], km, mask=sel)
        plsc.store_scatter(li, [r], im, mask=sel)

    pltpu.sync_copy(lv, shv.at[seqi], add=True)
    pltpu.sync_copy(li, shi.at[seqi], add=True)
    plsc.subcore_barrier()

    @pl.when(sid == 0)
    def _():
        pltpu.sync_copy(shv, lv)
        pltpu.sync_copy(lv, oval_hbm.at[pl.ds(pl.multiple_of(n * OUTP, L), OUTP)])
        pltpu.sync_copy(shi, li)
        pltpu.sync_copy(li, oidx_hbm.at[pl.ds(pl.multiple_of(n * OUTP, L), OUTP)])


def _sc_topk(p_flat, nbatch):
    mesh = plsc.VectorSubcoreMesh(core_axis_name="c", subcore_axis_name="s")
    i32 = jnp.int32
    out_type = [
        jax.ShapeDtypeStruct((nbatch * OUTP,), i32),
        jax.ShapeDtypeStruct((nbatch * OUTP,), i32),
    ]
    scratch = [
        pltpu.VMEM((CHUNK,), i32),
        pltpu.VMEM((NBIN,), i32),
        pltpu.VMEM((NSUB, SLABW), i32),
        pltpu.VMEM((NBIN,), i32),
        pltpu.VMEM((CAP_W,), i32),
        pltpu.VMEM((CAP_W,), i32),
        pltpu.VMEM((L,), i32),
        pltpu.VMEM((NSUB, 128), i32),
        pltpu.VMEM((CAP_G,), i32),
        pltpu.VMEM((CAP_G,), i32),
        pltpu.VMEM((OUTP,), i32),
        pltpu.VMEM((OUTP,), i32),
        pltpu.VMEM((OUTP,), i32),
        pltpu.VMEM((OUTP,), i32),
        pltpu.VMEM_SHARED((NSUB, NBIN), i32),
        pltpu.VMEM_SHARED((NBIN,), i32),
        pltpu.VMEM_SHARED((NBIN,), i32),
        pltpu.VMEM_SHARED((NBIN,), i32),
        pltpu.VMEM_SHARED((NSUB, 128), i32),
        pltpu.VMEM_SHARED((NSUB, CAP_W), i32),
        pltpu.VMEM_SHARED((NSUB, CAP_W), i32),
        pltpu.VMEM_SHARED((CAP_G,), i32),
        pltpu.VMEM_SHARED((CAP_G,), i32),
        pltpu.VMEM_SHARED((OUTP,), i32),
        pltpu.VMEM_SHARED((OUTP,), i32),
    ]
    fn = functools.partial(
        pl.kernel, mesh=mesh, out_type=out_type, scratch_types=scratch,
        compiler_params=pltpu.CompilerParams(needs_layout_passes=False),
    )(_sc_topk_body)
    return fn(p_flat)


def kernel(feature, W, b, gamma, beta):
    p = _conv_sigmoid(feature, W, b, gamma, beta)
    N = p.shape[0]
    p_keys = jax.lax.bitcast_convert_type(p.reshape(-1), jnp.int32)
    keys, idxs = _sc_topk(p_keys, N)
    values = jax.lax.bitcast_convert_type(
        keys.reshape(N, OUTP)[:, :TOPK], jnp.float32)
    indices = idxs.reshape(N, OUTP)[:, :TOPK]
    xy = jnp.stack(
        (
            (indices // W_).astype(jnp.float32) / float(H),
            (indices % W_).astype(jnp.float32) / float(W_),
        ),
        axis=-1,
    )
    return (p, values, xy, indices)
